# baseline (speedup 1.0000x reference)
import functools

import jax
import jax.numpy as jnp
from jax.experimental import pallas as pl
from jax.experimental.pallas import tpu as pltpu
from jax.experimental.pallas import tpu_sc as plsc

SIGMA_DIST = 0.1
BN_EPS = 1e-5
LEAKY_SLOPE = 0.1
BLK = 32
SC_WINDOW = 128
TBL_W = 128


def _sc_gather(table, idx_flat, nk_pad):
    mesh = plsc.VectorSubcoreMesh(core_axis_name="c", subcore_axis_name="s")
    n_workers = 32
    per_w = nk_pad // n_workers
    n_win = per_w // SC_WINDOW

    @functools.partial(
        pl.kernel,
        out_type=jax.ShapeDtypeStruct((nk_pad, TBL_W), jnp.int32),
        mesh=mesh,
        scratch_types=[
            pltpu.VMEM((per_w,), jnp.int32),
            pltpu.VMEM((SC_WINDOW, TBL_W), jnp.int32),
            pltpu.VMEM((SC_WINDOW, TBL_W), jnp.int32),
            pltpu.SemaphoreType.DMA,
            pltpu.SemaphoreType.DMA,
            pltpu.SemaphoreType.DMA,
            pltpu.SemaphoreType.DMA,
        ],
    )
    def gather_kernel(x_hbm, i_hbm, o_hbm, idx_v, rows_a, rows_b,
                      gsem_a, gsem_b, wsem_a, wsem_b):
        wid = jax.lax.axis_index("s") * 2 + jax.lax.axis_index("c")
        base = wid * per_w
        pltpu.sync_copy(i_hbm.at[pl.ds(base, per_w)], idx_v)

        def win_idx(g):
            return idx_v.at[pl.ds(g * SC_WINDOW, SC_WINDOW)]

        pltpu.async_copy(x_hbm.at[win_idx(0)], rows_a, gsem_a)
        pltpu.async_copy(x_hbm.at[win_idx(1)], rows_b, gsem_b)

        bufs = ((rows_a, gsem_a, wsem_a), (rows_b, gsem_b, wsem_b))

        @pl.loop(0, n_win, step=2)
        def _(g):
            for b in range(2):
                rows, gsem, wsem = bufs[b]
                gi = g + b
                pltpu.make_async_copy(x_hbm.at[win_idx(gi)], rows, gsem).wait()
                out_slc = o_hbm.at[pl.ds(base + gi * SC_WINDOW, SC_WINDOW)]
                pltpu.async_copy(rows, out_slc, wsem)
                nxt = gi + 2

                @pl.when(nxt < n_win)
                def _prefetch():
                    pltpu.make_async_copy(rows, out_slc, wsem).wait()
                    pltpu.async_copy(x_hbm.at[win_idx(nxt)], rows, gsem)

        for b in range(2):
            rows, _, wsem = bufs[b]
            gi = n_win - 2 + b
            out_slc = o_hbm.at[pl.ds(base + gi * SC_WINDOW, SC_WINDOW)]
            pltpu.make_async_copy(rows, out_slc, wsem).wait()

    return gather_kernel(table, idx_flat)


def _kpconv_block(f_ref, p_ref, posb_ref, w2_ref, kpt_ref, kp2_ref, e_ref,
                  out_ref, stats_ref, *, n_valid, kp, d, k):
    pid = pl.program_id(0)
    pos_own = posb_ref[:, 0:3]

    wsc = jnp.zeros((BLK, kp * d), jnp.float32)
    for kk in range(k):
        feat = f_ref[kk].astype(jnp.float32)
        rel = p_ref[kk][:, 0:3] - pos_own
        r2 = jnp.sum(rel * rel, axis=1, keepdims=True)
        acc = (rel[:, 0:1] * kpt_ref[0:1, :]
               + rel[:, 1:2] * kpt_ref[1:2, :]
               + rel[:, 2:3] * kpt_ref[2:3, :])
        d2 = jnp.maximum(r2 - 2.0 * acc + kp2_ref[0:1, :], 0.0)
        dist = jnp.sqrt(d2 + 1e-12)
        infl = jnp.maximum(0.0, 1.0 - dist * (1.0 / SIGMA_DIST))
        inflx = jnp.dot(infl, e_ref[...],
                        preferred_element_type=jnp.float32)
        featx = jnp.concatenate([feat] * kp, axis=1)
        wsc = wsc + inflx * featx

    x_blk = jnp.dot(wsc, w2_ref[...],
                    preferred_element_type=jnp.float32)
    out_ref[...] = x_blk

    row_ids = pid * BLK + jax.lax.broadcasted_iota(jnp.int32, (BLK, 1), 0)
    xm = jnp.where(row_ids < n_valid, x_blk, 0.0)
    sx = jnp.sum(xm, axis=0, keepdims=True)
    sq = jnp.sum(xm * xm, axis=0, keepdims=True)

    @pl.when(pid == 0)
    def _init():
        stats_ref[...] = jnp.zeros_like(stats_ref)

    stats_ref[0:1, :] += sx
    stats_ref[1:2, :] += sq


def _bn_lrelu(x_ref, stats_ref, gamma_ref, beta_ref, out_ref, *, n_valid):
    inv_n = 1.0 / n_valid
    mean = stats_ref[0:1, :] * inv_n
    var = stats_ref[1:2, :] * inv_n - mean * mean
    scale = gamma_ref[...] * jax.lax.rsqrt(var + BN_EPS)
    y = (x_ref[...] - mean) * scale + beta_ref[...]
    out_ref[...] = jnp.where(y >= 0.0, y, LEAKY_SLOPE * y)


@jax.jit
def kernel(points, point_features, neighbors, kernel_points, W, gamma, beta):
    n, d = point_features.shape
    k = neighbors.shape[1]
    kp, _, o = W.shape
    n_pad = ((n + 255) // 256) * 256
    nk_pad = n_pad * k

    pos = points[:, 1:4]
    h = d // 2
    fb = point_features.astype(jnp.bfloat16)
    f16 = jax.lax.bitcast_convert_type(fb.reshape(n, h, 2), jnp.uint16)
    packed = jax.lax.bitcast_convert_type(
        f16[:, :, 0].astype(jnp.uint32)
        | (f16[:, :, 1].astype(jnp.uint32) << 16),
        jnp.int32)
    posi = jax.lax.bitcast_convert_type(pos, jnp.int32)
    table = jnp.concatenate(
        [packed, posi, jnp.zeros((n, TBL_W - h - 3), jnp.int32)],
        axis=1)
    posb = jnp.pad(jnp.concatenate([pos, jnp.zeros((n, 1), jnp.float32)], 1),
                   ((0, n_pad - n), (0, 0)))
    idx_flat = jnp.pad(neighbors, ((0, n_pad - n), (0, 0))).T.reshape(-1)
    w2 = W.reshape(kp * d, o)
    kpt = kernel_points.T
    kp2 = jnp.sum(kernel_points * kernel_points, axis=1)[None, :]
    expand = jnp.kron(jnp.eye(kp, dtype=jnp.float32),
                      jnp.ones((1, d), jnp.float32))

    gat = _sc_gather(table, idx_flat, nk_pad)
    f3 = jax.lax.bitcast_convert_type(gat, jnp.bfloat16).reshape(
        k, n_pad, 2 * TBL_W)
    p3 = jax.lax.bitcast_convert_type(gat[:, h:h + 4], jnp.float32).reshape(
        k, n_pad, 4)

    grid = n_pad // BLK
    x_raw, stats = pl.pallas_call(
        functools.partial(_kpconv_block, n_valid=n, kp=kp, d=d, k=k),
        grid=(grid,),
        in_specs=[
            pl.BlockSpec((k, BLK, d), lambda i: (0, i, 0)),
            pl.BlockSpec((k, BLK, 4), lambda i: (0, i, 0)),
            pl.BlockSpec((BLK, 4), lambda i: (i, 0)),
            pl.BlockSpec((kp * d, o), lambda i: (0, 0)),
            pl.BlockSpec((3, kp), lambda i: (0, 0)),
            pl.BlockSpec((1, kp), lambda i: (0, 0)),
            pl.BlockSpec((kp, kp * d), lambda i: (0, 0)),
        ],
        out_specs=[
            pl.BlockSpec((BLK, o), lambda i: (i, 0)),
            pl.BlockSpec((8, o), lambda i: (0, 0)),
        ],
        out_shape=[
            jax.ShapeDtypeStruct((n_pad, o), jnp.float32),
            jax.ShapeDtypeStruct((8, o), jnp.float32),
        ],
    )(f3, p3, posb, w2, kpt, kp2, expand)

    out = pl.pallas_call(
        functools.partial(_bn_lrelu, n_valid=n),
        grid=(n_pad // 1280,),
        in_specs=[
            pl.BlockSpec((1280, o), lambda i: (i, 0)),
            pl.BlockSpec((8, o), lambda i: (0, 0)),
            pl.BlockSpec((1, o), lambda i: (0, 0)),
            pl.BlockSpec((1, o), lambda i: (0, 0)),
        ],
        out_specs=pl.BlockSpec((1280, o), lambda i: (i, 0)),
        out_shape=jax.ShapeDtypeStruct((n_pad, o), jnp.float32),
    )(x_raw, stats, gamma[None, :], beta[None, :])
    return out[:n]


# speedup vs baseline: 2.2857x; 2.2857x over previous
import functools

import jax
import jax.numpy as jnp
from jax.experimental import pallas as pl
from jax.experimental.pallas import tpu as pltpu
from jax.experimental.pallas import tpu_sc as plsc

SIGMA_DIST = 0.1
BN_EPS = 1e-5
LEAKY_SLOPE = 0.1
BLK = 32
SC_WINDOW = 128
TBL_W = 256


def _sc_gather(table, idx_flat, nk_pad):
    mesh = plsc.VectorSubcoreMesh(core_axis_name="c", subcore_axis_name="s")
    n_workers = 32
    per_w = nk_pad // n_workers
    n_win = per_w // SC_WINDOW

    @functools.partial(
        pl.kernel,
        out_type=jax.ShapeDtypeStruct((nk_pad, TBL_W), jnp.float32),
        mesh=mesh,
        scratch_types=[
            pltpu.VMEM((per_w,), jnp.int32),
            pltpu.VMEM((SC_WINDOW, TBL_W), jnp.float32),
            pltpu.VMEM((SC_WINDOW, TBL_W), jnp.float32),
            pltpu.SemaphoreType.DMA,
            pltpu.SemaphoreType.DMA,
            pltpu.SemaphoreType.DMA,
            pltpu.SemaphoreType.DMA,
        ],
    )
    def gather_kernel(x_hbm, i_hbm, o_hbm, idx_v, rows_a, rows_b,
                      gsem_a, gsem_b, wsem_a, wsem_b):
        wid = jax.lax.axis_index("s") * 2 + jax.lax.axis_index("c")
        base = wid * per_w
        pltpu.sync_copy(i_hbm.at[pl.ds(base, per_w)], idx_v)

        def win_idx(g):
            return idx_v.at[pl.ds(g * SC_WINDOW, SC_WINDOW)]

        pltpu.async_copy(x_hbm.at[win_idx(0)], rows_a, gsem_a)
        pltpu.async_copy(x_hbm.at[win_idx(1)], rows_b, gsem_b)

        bufs = ((rows_a, gsem_a, wsem_a), (rows_b, gsem_b, wsem_b))

        @pl.loop(0, n_win, step=2)
        def _(g):
            for b in range(2):
                rows, gsem, wsem = bufs[b]
                gi = g + b
                pltpu.make_async_copy(x_hbm.at[win_idx(gi)], rows, gsem).wait()
                out_slc = o_hbm.at[pl.ds(base + gi * SC_WINDOW, SC_WINDOW)]
                pltpu.async_copy(rows, out_slc, wsem)
                nxt = gi + 2

                @pl.when(nxt < n_win)
                def _prefetch():
                    pltpu.make_async_copy(rows, out_slc, wsem).wait()
                    pltpu.async_copy(x_hbm.at[win_idx(nxt)], rows, gsem)

        for b in range(2):
            rows, _, wsem = bufs[b]
            gi = n_win - 2 + b
            out_slc = o_hbm.at[pl.ds(base + gi * SC_WINDOW, SC_WINDOW)]
            pltpu.make_async_copy(rows, out_slc, wsem).wait()

    return gather_kernel(table, idx_flat)


def _kpconv_block(g_ref, posb_ref, w2_ref, kpt_ref, kp2_ref, e_ref,
                  out_ref, stats_ref, *, n_valid, kp, d, k):
    pid = pl.program_id(0)
    pos_own = posb_ref[:, 0:3]

    wsc = jnp.zeros((BLK, kp * d), jnp.float32)
    for kk in range(k):
        rows = g_ref[kk]
        feat = rows[:, 0:d]
        rel = rows[:, d:d + 3] - pos_own
        r2 = jnp.sum(rel * rel, axis=1, keepdims=True)
        acc = (rel[:, 0:1] * kpt_ref[0:1, :]
               + rel[:, 1:2] * kpt_ref[1:2, :]
               + rel[:, 2:3] * kpt_ref[2:3, :])
        d2 = jnp.maximum(r2 - 2.0 * acc + kp2_ref[0:1, :], 0.0)
        dist = jnp.sqrt(d2 + 1e-12)
        infl = jnp.maximum(0.0, 1.0 - dist * (1.0 / SIGMA_DIST))
        inflx = jnp.dot(infl, e_ref[...],
                        preferred_element_type=jnp.float32)
        featx = jnp.concatenate([feat] * kp, axis=1)
        wsc = wsc + inflx * featx

    x_blk = jnp.dot(wsc, w2_ref[...],
                    preferred_element_type=jnp.float32)
    out_ref[...] = x_blk

    row_ids = pid * BLK + jax.lax.broadcasted_iota(jnp.int32, (BLK, 1), 0)
    xm = jnp.where(row_ids < n_valid, x_blk, 0.0)
    sx = jnp.sum(xm, axis=0, keepdims=True)
    sq = jnp.sum(xm * xm, axis=0, keepdims=True)

    @pl.when(pid == 0)
    def _init():
        stats_ref[...] = jnp.zeros_like(stats_ref)

    stats_ref[0:1, :] += sx
    stats_ref[1:2, :] += sq


def _bn_lrelu(x_ref, stats_ref, gamma_ref, beta_ref, out_ref, *, n_valid):
    inv_n = 1.0 / n_valid
    mean = stats_ref[0:1, :] * inv_n
    var = stats_ref[1:2, :] * inv_n - mean * mean
    scale = gamma_ref[...] * jax.lax.rsqrt(var + BN_EPS)
    y = (x_ref[...] - mean) * scale + beta_ref[...]
    out_ref[...] = jnp.where(y >= 0.0, y, LEAKY_SLOPE * y)


@jax.jit
def kernel(points, point_features, neighbors, kernel_points, W, gamma, beta):
    n, d = point_features.shape
    k = neighbors.shape[1]
    kp, _, o = W.shape
    n_pad = ((n + 255) // 256) * 256
    nk_pad = n_pad * k

    pos = points[:, 1:4]
    table = jnp.concatenate(
        [point_features, pos, jnp.zeros((n, TBL_W - d - 3), jnp.float32)],
        axis=1)
    posb = jnp.pad(jnp.concatenate([pos, jnp.zeros((n, 1), jnp.float32)], 1),
                   ((0, n_pad - n), (0, 0)))
    idx_flat = jnp.pad(neighbors, ((0, n_pad - n), (0, 0))).T.reshape(-1)
    w2 = W.reshape(kp * d, o)
    kpt = kernel_points.T
    kp2 = jnp.sum(kernel_points * kernel_points, axis=1)[None, :]
    expand = jnp.kron(jnp.eye(kp, dtype=jnp.float32),
                      jnp.ones((1, d), jnp.float32))

    gathered = _sc_gather(table, idx_flat, nk_pad)
    g3 = gathered.reshape(k, n_pad, TBL_W)

    grid = n_pad // BLK
    x_raw, stats = pl.pallas_call(
        functools.partial(_kpconv_block, n_valid=n, kp=kp, d=d, k=k),
        grid=(grid,),
        in_specs=[
            pl.BlockSpec((k, BLK, TBL_W), lambda i: (0, i, 0)),
            pl.BlockSpec((BLK, 4), lambda i: (i, 0)),
            pl.BlockSpec((kp * d, o), lambda i: (0, 0)),
            pl.BlockSpec((3, kp), lambda i: (0, 0)),
            pl.BlockSpec((1, kp), lambda i: (0, 0)),
            pl.BlockSpec((kp, kp * d), lambda i: (0, 0)),
        ],
        out_specs=[
            pl.BlockSpec((BLK, o), lambda i: (i, 0)),
            pl.BlockSpec((8, o), lambda i: (0, 0)),
        ],
        out_shape=[
            jax.ShapeDtypeStruct((n_pad, o), jnp.float32),
            jax.ShapeDtypeStruct((8, o), jnp.float32),
        ],
    )(g3, posb, w2, kpt, kp2, expand)

    out = pl.pallas_call(
        functools.partial(_bn_lrelu, n_valid=n),
        grid=(n_pad // 1280,),
        in_specs=[
            pl.BlockSpec((1280, o), lambda i: (i, 0)),
            pl.BlockSpec((8, o), lambda i: (0, 0)),
            pl.BlockSpec((1, o), lambda i: (0, 0)),
            pl.BlockSpec((1, o), lambda i: (0, 0)),
        ],
        out_specs=pl.BlockSpec((1280, o), lambda i: (i, 0)),
        out_shape=jax.ShapeDtypeStruct((n_pad, o), jnp.float32),
    )(x_raw, stats, gamma[None, :], beta[None, :])
    return out[:n]
